# baseline (speedup 1.0000x reference)
import jax
import jax.numpy as jnp
from jax import lax
from jax.experimental import pallas as pl
from jax.experimental.pallas import tpu as pltpu

T = 2048
HIDDEN = 2048
FFN = 8192
E = 8
K = 2
CAP = 640

BF = 512
NF = FFN // BF


def _routing_body(x_ref, wg_ref, p_ref, g_ref):
    x = x_ref[...]
    wg = wg_ref[...]
    logits = jnp.dot(x, wg, preferred_element_type=jnp.float32)
    m = jnp.max(logits, axis=-1, keepdims=True)
    ex = jnp.exp(logits - m)
    probs = ex / jnp.sum(ex, axis=-1, keepdims=True)

    lane = lax.broadcasted_iota(jnp.int32, (T, E), 1)
    m1 = jnp.max(probs, axis=-1, keepdims=True)
    idx1 = jnp.min(jnp.where(probs == m1, lane, E), axis=-1, keepdims=True)
    oh0 = (lane == idx1).astype(jnp.float32)
    probs2 = jnp.where(lane == idx1, -1e30, probs)
    m2 = jnp.max(probs2, axis=-1, keepdims=True)
    idx2 = jnp.min(jnp.where(probs2 == m2, lane, E), axis=-1, keepdims=True)
    oh1 = (lane == idx2).astype(jnp.float32)

    s = m1 + m2
    g1 = m1 / s
    g2 = m2 / s

    row = lax.broadcasted_iota(jnp.int32, (T, T), 0)
    col = lax.broadcasted_iota(jnp.int32, (T, T), 1)
    tri = (col <= row).astype(jnp.float32)
    c0 = jnp.dot(tri, oh0, preferred_element_type=jnp.float32)
    c1 = jnp.dot(tri, oh1, preferred_element_type=jnp.float32)

    pos0 = jnp.sum(c0 * oh0, axis=-1, keepdims=True) - 1.0
    counts0 = jnp.sum(oh0, axis=0, keepdims=True)
    pos1 = (jnp.sum(c1 * oh1, axis=-1, keepdims=True) - 1.0
            + jnp.sum(counts0 * oh1, axis=-1, keepdims=True))
    keep0 = (pos0 < CAP).astype(jnp.float32)
    keep1 = (pos1 < CAP).astype(jnp.float32)

    p_ref[...] = oh0 * (pos0 + 1.0) * keep0 + oh1 * (pos1 + 1.0) * keep1 - 1.0
    g_ref[...] = oh0 * g1 + oh1 * g2


def _p_column(p_ref, e):
    lane_e = lax.broadcasted_iota(jnp.int32, (T, E), 1)
    return jnp.sum(jnp.where(lane_e == e, p_ref[...], 0.0), axis=-1,
                   keepdims=True).astype(jnp.int32)


def _dispatch_body(p_ref, x_ref, ein_ref):
    e = pl.program_id(0)
    p_col = _p_column(p_ref, e)
    cap_lane = lax.broadcasted_iota(jnp.int32, (T, CAP), 1)
    disp = (cap_lane == p_col).astype(jnp.float32)
    ein_ref[...] = lax.dot_general(
        disp, x_ref[...], (((0,), (0,)), ((), ())),
        preferred_element_type=jnp.float32)[None]


def _ffn_body(ein_ref, w1_ref, b1_ref, w2_ref, b2_ref, eo_ref, eo_scr):
    f = pl.program_id(1)
    ein = ein_ref[...].reshape(CAP, HIDDEN).astype(jnp.bfloat16)
    w1 = w1_ref[...].reshape(HIDDEN, BF).astype(jnp.bfloat16)
    b1 = b1_ref[...].reshape(1, BF)
    h = jnp.maximum(
        jnp.dot(ein, w1, preferred_element_type=jnp.float32) + b1, 0.0)
    w2 = w2_ref[...].reshape(BF, HIDDEN).astype(jnp.bfloat16)
    part = jnp.dot(h.astype(jnp.bfloat16), w2,
                   preferred_element_type=jnp.float32)

    @pl.when(f == 0)
    def _init():
        eo_scr[...] = part

    @pl.when(f > 0)
    def _acc():
        eo_scr[...] += part

    @pl.when(f == NF - 1)
    def _write():
        b2 = b2_ref[...].reshape(1, HIDDEN)
        eo_ref[...] = (eo_scr[...] + b2)[None]


def _combine_body(p_ref, g_ref, eo_ref, out_ref):
    e = pl.program_id(0)
    p_col = _p_column(p_ref, e)
    cap_lane = lax.broadcasted_iota(jnp.int32, (T, CAP), 1)
    lane_e = lax.broadcasted_iota(jnp.int32, (T, E), 1)
    g_col = jnp.sum(jnp.where(lane_e == e, g_ref[...], 0.0), axis=-1,
                    keepdims=True)
    cg = jnp.where(cap_lane == p_col, g_col, 0.0)
    eo = eo_ref[...].reshape(CAP, HIDDEN)
    contrib = jnp.dot(cg, eo, preferred_element_type=jnp.float32)

    @pl.when(e == 0)
    def _init():
        out_ref[...] = contrib

    @pl.when(e > 0)
    def _acc():
        out_ref[...] += contrib


@jax.jit
def kernel(x, Wg, W1, b1, W2, b2):
    p_map, g_map = pl.pallas_call(
        _routing_body,
        out_shape=(
            jax.ShapeDtypeStruct((T, E), jnp.float32),
            jax.ShapeDtypeStruct((T, E), jnp.float32),
        ),
    )(x, Wg)

    ein = pl.pallas_call(
        _dispatch_body,
        grid=(E,),
        in_specs=[
            pl.BlockSpec((T, E), lambda e: (0, 0)),
            pl.BlockSpec((T, HIDDEN), lambda e: (0, 0)),
        ],
        out_specs=pl.BlockSpec((1, CAP, HIDDEN), lambda e: (e, 0, 0)),
        out_shape=jax.ShapeDtypeStruct((E, CAP, HIDDEN), jnp.float32),
    )(p_map, x)

    b1r = b1.reshape(E, 1, FFN)
    b2r = b2.reshape(E, 1, HIDDEN)

    eo = pl.pallas_call(
        _ffn_body,
        grid=(E, NF),
        in_specs=[
            pl.BlockSpec((1, CAP, HIDDEN), lambda e, f: (e, 0, 0)),
            pl.BlockSpec((1, HIDDEN, BF), lambda e, f: (e, 0, f)),
            pl.BlockSpec((1, 1, BF), lambda e, f: (e, 0, f)),
            pl.BlockSpec((1, BF, HIDDEN), lambda e, f: (e, f, 0)),
            pl.BlockSpec((1, 1, HIDDEN), lambda e, f: (e, 0, 0)),
        ],
        out_specs=pl.BlockSpec((1, CAP, HIDDEN), lambda e, f: (e, 0, 0)),
        out_shape=jax.ShapeDtypeStruct((E, CAP, HIDDEN), jnp.float32),
        scratch_shapes=[pltpu.VMEM((CAP, HIDDEN), jnp.float32)],
    )(ein, W1, b1r, W2, b2r)

    out = pl.pallas_call(
        _combine_body,
        grid=(E,),
        in_specs=[
            pl.BlockSpec((T, E), lambda e: (0, 0)),
            pl.BlockSpec((T, E), lambda e: (0, 0)),
            pl.BlockSpec((1, CAP, HIDDEN), lambda e: (e, 0, 0)),
        ],
        out_specs=pl.BlockSpec((T, HIDDEN), lambda e: (0, 0)),
        out_shape=jax.ShapeDtypeStruct((T, HIDDEN), jnp.float32),
    )(p_map, g_map, eo)
    return out


# speedup vs baseline: 1.0989x; 1.0989x over previous
import jax
import jax.numpy as jnp
from jax import lax
from jax.experimental import pallas as pl
from jax.experimental.pallas import tpu as pltpu

T = 2048
HIDDEN = 2048
FFN = 8192
E = 8
K = 2
CAP = 640

BF = 1024
NF = FFN // BF


def _routing_body(x_ref, wg_ref, p_ref, g_ref):
    x = x_ref[...]
    wg = wg_ref[...]
    logits = jnp.dot(x, wg, preferred_element_type=jnp.float32)
    m = jnp.max(logits, axis=-1, keepdims=True)
    ex = jnp.exp(logits - m)
    probs = ex / jnp.sum(ex, axis=-1, keepdims=True)

    lane = lax.broadcasted_iota(jnp.int32, (T, E), 1)
    m1 = jnp.max(probs, axis=-1, keepdims=True)
    idx1 = jnp.min(jnp.where(probs == m1, lane, E), axis=-1, keepdims=True)
    oh0 = (lane == idx1).astype(jnp.float32)
    probs2 = jnp.where(lane == idx1, -1e30, probs)
    m2 = jnp.max(probs2, axis=-1, keepdims=True)
    idx2 = jnp.min(jnp.where(probs2 == m2, lane, E), axis=-1, keepdims=True)
    oh1 = (lane == idx2).astype(jnp.float32)

    s = m1 + m2
    g1 = m1 / s
    g2 = m2 / s

    row = lax.broadcasted_iota(jnp.int32, (T, T), 0)
    col = lax.broadcasted_iota(jnp.int32, (T, T), 1)
    tri = (col <= row).astype(jnp.float32)
    c0 = jnp.dot(tri, oh0, preferred_element_type=jnp.float32)
    c1 = jnp.dot(tri, oh1, preferred_element_type=jnp.float32)

    pos0 = jnp.sum(c0 * oh0, axis=-1, keepdims=True) - 1.0
    counts0 = jnp.sum(oh0, axis=0, keepdims=True)
    pos1 = (jnp.sum(c1 * oh1, axis=-1, keepdims=True) - 1.0
            + jnp.sum(counts0 * oh1, axis=-1, keepdims=True))
    keep0 = (pos0 < CAP).astype(jnp.float32)
    keep1 = (pos1 < CAP).astype(jnp.float32)

    p_ref[...] = oh0 * (pos0 + 1.0) * keep0 + oh1 * (pos1 + 1.0) * keep1 - 1.0
    g_ref[...] = oh0 * g1 + oh1 * g2


def _p_column(p_ref, e):
    lane_e = lax.broadcasted_iota(jnp.int32, (T, E), 1)
    return jnp.sum(jnp.where(lane_e == e, p_ref[...], 0.0), axis=-1,
                   keepdims=True).astype(jnp.int32)


def _dispatch_body(p_ref, x_ref, ein_ref):
    e = pl.program_id(0)
    p_col = _p_column(p_ref, e)
    cap_lane = lax.broadcasted_iota(jnp.int32, (T, CAP), 1)
    disp = (cap_lane == p_col).astype(jnp.bfloat16)
    ein = lax.dot_general(
        disp, x_ref[...].astype(jnp.bfloat16), (((0,), (0,)), ((), ())),
        preferred_element_type=jnp.float32)
    ein_ref[...] = ein.astype(jnp.bfloat16)[None]


def _ffn_body(ein_ref, w1_ref, b1_ref, w2_ref, b2_ref, eo_ref, eo_scr):
    f = pl.program_id(1)
    ein = ein_ref[...].reshape(CAP, HIDDEN)
    w1 = w1_ref[...].reshape(HIDDEN, BF).astype(jnp.bfloat16)
    b1 = b1_ref[...].reshape(1, BF)
    h = jnp.maximum(
        jnp.dot(ein, w1, preferred_element_type=jnp.float32) + b1, 0.0)
    w2 = w2_ref[...].reshape(BF, HIDDEN).astype(jnp.bfloat16)
    part = jnp.dot(h.astype(jnp.bfloat16), w2,
                   preferred_element_type=jnp.float32)

    @pl.when(f == 0)
    def _init():
        eo_scr[...] = part

    @pl.when(f > 0)
    def _acc():
        eo_scr[...] += part

    @pl.when(f == NF - 1)
    def _write():
        b2 = b2_ref[...].reshape(1, HIDDEN)
        eo_ref[...] = (eo_scr[...] + b2)[None]


def _combine_body(p_ref, g_ref, eo_ref, out_ref):
    e = pl.program_id(0)
    p_col = _p_column(p_ref, e)
    cap_lane = lax.broadcasted_iota(jnp.int32, (T, CAP), 1)
    lane_e = lax.broadcasted_iota(jnp.int32, (T, E), 1)
    g_col = jnp.sum(jnp.where(lane_e == e, g_ref[...], 0.0), axis=-1,
                    keepdims=True)
    cg = jnp.where(cap_lane == p_col, g_col, 0.0)
    eo = eo_ref[...].reshape(CAP, HIDDEN)
    contrib = jnp.dot(cg, eo, preferred_element_type=jnp.float32)

    @pl.when(e == 0)
    def _init():
        out_ref[...] = contrib

    @pl.when(e > 0)
    def _acc():
        out_ref[...] += contrib


@jax.jit
def kernel(x, Wg, W1, b1, W2, b2):
    p_map, g_map = pl.pallas_call(
        _routing_body,
        out_shape=(
            jax.ShapeDtypeStruct((T, E), jnp.float32),
            jax.ShapeDtypeStruct((T, E), jnp.float32),
        ),
    )(x, Wg)

    ein = pl.pallas_call(
        _dispatch_body,
        grid=(E,),
        in_specs=[
            pl.BlockSpec((T, E), lambda e: (0, 0)),
            pl.BlockSpec((T, HIDDEN), lambda e: (0, 0)),
        ],
        out_specs=pl.BlockSpec((1, CAP, HIDDEN), lambda e: (e, 0, 0)),
        out_shape=jax.ShapeDtypeStruct((E, CAP, HIDDEN), jnp.bfloat16),
    )(p_map, x)

    b1r = b1.reshape(E, 1, FFN)
    b2r = b2.reshape(E, 1, HIDDEN)

    eo = pl.pallas_call(
        _ffn_body,
        grid=(E, NF),
        in_specs=[
            pl.BlockSpec((1, CAP, HIDDEN), lambda e, f: (e, 0, 0)),
            pl.BlockSpec((1, HIDDEN, BF), lambda e, f: (e, 0, f)),
            pl.BlockSpec((1, 1, BF), lambda e, f: (e, 0, f)),
            pl.BlockSpec((1, BF, HIDDEN), lambda e, f: (e, f, 0)),
            pl.BlockSpec((1, 1, HIDDEN), lambda e, f: (e, 0, 0)),
        ],
        out_specs=pl.BlockSpec((1, CAP, HIDDEN), lambda e, f: (e, 0, 0)),
        out_shape=jax.ShapeDtypeStruct((E, CAP, HIDDEN), jnp.float32),
        scratch_shapes=[pltpu.VMEM((CAP, HIDDEN), jnp.float32)],
    )(ein, W1, b1r, W2, b2r)

    out = pl.pallas_call(
        _combine_body,
        grid=(E,),
        in_specs=[
            pl.BlockSpec((T, E), lambda e: (0, 0)),
            pl.BlockSpec((T, E), lambda e: (0, 0)),
            pl.BlockSpec((1, CAP, HIDDEN), lambda e: (e, 0, 0)),
        ],
        out_specs=pl.BlockSpec((T, HIDDEN), lambda e: (0, 0)),
        out_shape=jax.ShapeDtypeStruct((T, HIDDEN), jnp.float32),
    )(p_map, g_map, eo)
    return out


# speedup vs baseline: 1.0991x; 1.0001x over previous
import jax
import jax.numpy as jnp
from jax import lax
from jax.experimental import pallas as pl
from jax.experimental.pallas import tpu as pltpu

T = 2048
HIDDEN = 2048
FFN = 8192
E = 8
K = 2
CAP = 640

BF = 1024
NF = FFN // BF


def _routing_body(x_ref, wg_ref, p_ref, g_ref):
    x = x_ref[...]
    wg = wg_ref[...]
    logits = jnp.dot(x, wg, preferred_element_type=jnp.float32)
    m = jnp.max(logits, axis=-1, keepdims=True)
    ex = jnp.exp(logits - m)
    probs = ex / jnp.sum(ex, axis=-1, keepdims=True)

    lane = lax.broadcasted_iota(jnp.int32, (T, E), 1)
    m1 = jnp.max(probs, axis=-1, keepdims=True)
    idx1 = jnp.min(jnp.where(probs == m1, lane, E), axis=-1, keepdims=True)
    oh0 = (lane == idx1).astype(jnp.float32)
    probs2 = jnp.where(lane == idx1, -1e30, probs)
    m2 = jnp.max(probs2, axis=-1, keepdims=True)
    idx2 = jnp.min(jnp.where(probs2 == m2, lane, E), axis=-1, keepdims=True)
    oh1 = (lane == idx2).astype(jnp.float32)

    s = m1 + m2
    g1 = m1 / s
    g2 = m2 / s

    row = lax.broadcasted_iota(jnp.int32, (T, T), 0)
    col = lax.broadcasted_iota(jnp.int32, (T, T), 1)
    tri = (col <= row).astype(jnp.float32)
    c0 = jnp.dot(tri, oh0, preferred_element_type=jnp.float32)
    c1 = jnp.dot(tri, oh1, preferred_element_type=jnp.float32)

    pos0 = jnp.sum(c0 * oh0, axis=-1, keepdims=True) - 1.0
    counts0 = jnp.sum(oh0, axis=0, keepdims=True)
    pos1 = (jnp.sum(c1 * oh1, axis=-1, keepdims=True) - 1.0
            + jnp.sum(counts0 * oh1, axis=-1, keepdims=True))
    keep0 = (pos0 < CAP).astype(jnp.float32)
    keep1 = (pos1 < CAP).astype(jnp.float32)

    p_ref[...] = oh0 * (pos0 + 1.0) * keep0 + oh1 * (pos1 + 1.0) * keep1 - 1.0
    g_ref[...] = oh0 * g1 + oh1 * g2


def _p_column(p_ref, e):
    lane_e = lax.broadcasted_iota(jnp.int32, (T, E), 1)
    return jnp.sum(jnp.where(lane_e == e, p_ref[...], 0.0), axis=-1,
                   keepdims=True).astype(jnp.int32)


def _dispatch_body(p_ref, x_ref, ein_ref):
    e = pl.program_id(0)
    p_col = _p_column(p_ref, e)
    cap_lane = lax.broadcasted_iota(jnp.int32, (T, CAP), 1)
    disp = (cap_lane == p_col).astype(jnp.float32)
    ein = lax.dot_general(
        disp, x_ref[...], (((0,), (0,)), ((), ())),
        preferred_element_type=jnp.float32)
    ein_ref[...] = ein[None]


def _ffn_body(ein_ref, w1_ref, b1_ref, w2_ref, b2_ref, eo_ref, eo_scr):
    f = pl.program_id(1)
    ein = ein_ref[...].reshape(CAP, HIDDEN)
    w1 = w1_ref[...].reshape(HIDDEN, BF)
    b1 = b1_ref[...].reshape(1, BF)
    h = jnp.maximum(
        jnp.dot(ein, w1, preferred_element_type=jnp.float32) + b1, 0.0)
    w2 = w2_ref[...].reshape(BF, HIDDEN)
    part = jnp.dot(h, w2, preferred_element_type=jnp.float32)

    @pl.when(f == 0)
    def _init():
        eo_scr[...] = part

    @pl.when(f > 0)
    def _acc():
        eo_scr[...] += part

    @pl.when(f == NF - 1)
    def _write():
        b2 = b2_ref[...].reshape(1, HIDDEN)
        eo_ref[...] = (eo_scr[...] + b2).astype(jnp.bfloat16)[None]


def _combine_body(p_ref, g_ref, eo_ref, out_ref):
    e = pl.program_id(0)
    p_col = _p_column(p_ref, e)
    cap_lane = lax.broadcasted_iota(jnp.int32, (T, CAP), 1)
    lane_e = lax.broadcasted_iota(jnp.int32, (T, E), 1)
    g_col = jnp.sum(jnp.where(lane_e == e, g_ref[...], 0.0), axis=-1,
                    keepdims=True)
    cg = jnp.where(cap_lane == p_col, g_col, 0.0).astype(jnp.bfloat16)
    eo = eo_ref[...].reshape(CAP, HIDDEN)
    contrib = jnp.dot(cg, eo, preferred_element_type=jnp.float32)

    @pl.when(e == 0)
    def _init():
        out_ref[...] = contrib

    @pl.when(e > 0)
    def _acc():
        out_ref[...] += contrib


@jax.jit
def kernel(x, Wg, W1, b1, W2, b2):
    p_map, g_map = pl.pallas_call(
        _routing_body,
        out_shape=(
            jax.ShapeDtypeStruct((T, E), jnp.float32),
            jax.ShapeDtypeStruct((T, E), jnp.float32),
        ),
    )(x, Wg)

    ein = pl.pallas_call(
        _dispatch_body,
        grid=(E,),
        in_specs=[
            pl.BlockSpec((T, E), lambda e: (0, 0)),
            pl.BlockSpec((T, HIDDEN), lambda e: (0, 0)),
        ],
        out_specs=pl.BlockSpec((1, CAP, HIDDEN), lambda e: (e, 0, 0)),
        out_shape=jax.ShapeDtypeStruct((E, CAP, HIDDEN), jnp.float32),
    )(p_map, x)

    b1r = b1.reshape(E, 1, FFN)
    b2r = b2.reshape(E, 1, HIDDEN)

    eo = pl.pallas_call(
        _ffn_body,
        grid=(E, NF),
        in_specs=[
            pl.BlockSpec((1, CAP, HIDDEN), lambda e, f: (e, 0, 0)),
            pl.BlockSpec((1, HIDDEN, BF), lambda e, f: (e, 0, f)),
            pl.BlockSpec((1, 1, BF), lambda e, f: (e, 0, f)),
            pl.BlockSpec((1, BF, HIDDEN), lambda e, f: (e, f, 0)),
            pl.BlockSpec((1, 1, HIDDEN), lambda e, f: (e, 0, 0)),
        ],
        out_specs=pl.BlockSpec((1, CAP, HIDDEN), lambda e, f: (e, 0, 0)),
        out_shape=jax.ShapeDtypeStruct((E, CAP, HIDDEN), jnp.bfloat16),
        scratch_shapes=[pltpu.VMEM((CAP, HIDDEN), jnp.float32)],
    )(ein, W1, b1r, W2, b2r)

    out = pl.pallas_call(
        _combine_body,
        grid=(E,),
        in_specs=[
            pl.BlockSpec((T, E), lambda e: (0, 0)),
            pl.BlockSpec((T, E), lambda e: (0, 0)),
            pl.BlockSpec((1, CAP, HIDDEN), lambda e: (e, 0, 0)),
        ],
        out_specs=pl.BlockSpec((T, HIDDEN), lambda e: (0, 0)),
        out_shape=jax.ShapeDtypeStruct((T, HIDDEN), jnp.float32),
    )(p_map, g_map, eo)
    return out
